# baseline (speedup 1.0000x reference)
import functools
import jax
import jax.numpy as jnp
from jax import lax
from jax.experimental import pallas as pl
from jax.experimental.pallas import tpu as pltpu

N_NODES_C = 10000
NUM_GRAPHS_C = 8
H_C = 64
NEG_INF = float("-inf")
CH = 1000
NCH = N_NODES_C // CH

_HIGH = lax.Precision.HIGHEST


def _dot(a, b):
    return lax.dot_general(a, b, (((1,), (0,)), ((), ())),
                           precision=_HIGH, preferred_element_type=jnp.float32)


def _dotp(a, b):
    return lax.dot_general(a, b, (((1,), (0,)), ((), ())),
                           preferred_element_type=jnp.float32)


def _dot_t(a, b):
    return lax.dot_general(a, b, (((0,), (0,)), ((), ())),
                           precision=_HIGH, preferred_element_type=jnp.float32)


def _lrelu(v):
    return jnp.where(v > 0, v, 0.01 * v)


def _rows(i):
    return pl.ds(pl.multiple_of(i * CH, CH), CH)


def _chunk_loop(body, init):
    return lax.fori_loop(0, NCH, lambda i, c: body(_rows(i), c), init)


def _mask_at(batch_ref, ds):
    iota = lax.broadcasted_iota(jnp.int32, (CH, NUM_GRAPHS_C), 1)
    return (batch_ref[ds, :] == iota).astype(jnp.float32)


def _count(batch_ref):
    ones = jnp.ones((CH, 1), jnp.float32)

    def body(ds, cnt):
        return cnt + _dot_t(_mask_at(batch_ref, ds), ones)

    cnt = _chunk_loop(body, jnp.zeros((NUM_GRAPHS_C, 1), jnp.float32))
    return jnp.maximum(cnt, 1.0)


def _norm_lrelu(batch_ref, cnt, dst_ref, cols, width):
    def b1(ds, sums):
        return sums + _dot_t(_mask_at(batch_ref, ds), dst_ref[ds, cols])

    sums = _chunk_loop(b1, jnp.zeros((NUM_GRAPHS_C, width), jnp.float32))
    mean = sums / cnt

    def b2(ds, var):
        m = _mask_at(batch_ref, ds)
        hc = dst_ref[ds, cols] - _dot(m, mean)
        dst_ref[ds, cols] = hc
        return var + _dot_t(m, hc * hc)

    var = _chunk_loop(b2, jnp.zeros((NUM_GRAPHS_C, width), jnp.float32))
    rs = lax.rsqrt(var / cnt + 1e-5)

    def b3(ds, c):
        m = _mask_at(batch_ref, ds)
        dst_ref[ds, cols] = _lrelu(dst_ref[ds, cols] * _dot(m, rs))
        return c

    _chunk_loop(b3, 0)


def _lin_seq_ref(batch_ref, cnt, src_ref, src_cols, W, b, dst_ref, dst_cols,
                 width):
    def body(ds, c):
        dst_ref[ds, dst_cols] = _dotp(src_ref[ds, src_cols], W) + b
        return c

    _chunk_loop(body, 0)
    _norm_lrelu(batch_ref, cnt, dst_ref, dst_cols, width)


def _seg_max_cols(batch_ref, src_ref, cols, width):
    def body(ds, gm):
        m = _mask_at(batch_ref, ds)
        v = src_ref[ds, cols]
        parts = []
        for g in range(NUM_GRAPHS_C):
            sel = m[:, g:g + 1] > 0
            parts.append(jnp.max(jnp.where(sel, v, NEG_INF), axis=0,
                                 keepdims=True))
        return jnp.maximum(gm, jnp.concatenate(parts, axis=0))

    return _chunk_loop(
        body, jnp.full((NUM_GRAPHS_C, width), NEG_INF, jnp.float32))


_ALL = slice(None)
_LO = slice(0, H_C)
_HI = slice(H_C, 2 * H_C)


def _tc1_body(x_ref, batch_ref, W0_ref, b0_ref, Ws1_ref, bs1_ref, Ws2_ref,
              bs2_ref, Ws3_ref, bs3_ref, Wf1_ref, bf1_ref, Wf2_ref, bf2_ref,
              Wf3_ref, bf3_ref, We1_ref, be1_ref, AB_ref,
              x0t1_ref, t2_ref, t3_ref):
    cnt = _count(batch_ref)
    _lin_seq_ref(batch_ref, cnt, x_ref, _ALL, W0_ref[:], b0_ref[:],
                 x0t1_ref, _LO, H_C)
    _lin_seq_ref(batch_ref, cnt, x0t1_ref, _LO, Ws1_ref[:], bs1_ref[:],
                 x0t1_ref, _HI, 64)
    _lin_seq_ref(batch_ref, cnt, x0t1_ref, _HI, Ws2_ref[:], bs2_ref[:],
                 t2_ref, _ALL, 128)
    _lin_seq_ref(batch_ref, cnt, t2_ref, _ALL, Ws3_ref[:], bs3_ref[:],
                 t3_ref, _ALL, 256)
    tp = _seg_max_cols(batch_ref, t3_ref, _ALL, 256)
    tp = jnp.where(jnp.isfinite(tp), tp, 0.0)
    tp = _lrelu(_dotp(tp, Wf1_ref[:]) + bf1_ref[:])
    tp = _lrelu(_dotp(tp, Wf2_ref[:]) + bf2_ref[:])
    tp = _dotp(tp, Wf3_ref[:]) + bf3_ref[:]
    eye_flat = (lax.broadcasted_iota(jnp.int32, (1, H_C * H_C), 1) % (H_C + 1)
                == 0).astype(jnp.float32)
    trans = (tp + eye_flat).reshape(NUM_GRAPHS_C, H_C, H_C)
    We = We1_ref[:]

    def bT(ds, c):
        x0c = x0t1_ref[ds, _LO]
        m = _mask_at(batch_ref, ds)
        xT = jnp.zeros((CH, H_C), jnp.float32)
        for g in range(NUM_GRAPHS_C):
            xT = xT + m[:, g:g + 1] * _dotp(x0c, trans[g])
        q = _dotp(xT, We[H_C:, :])
        AB_ref[ds, _LO] = _dotp(xT, We[:H_C, :]) - q + be1_ref[:]
        AB_ref[ds, _HI] = q
        return c

    _chunk_loop(bT, 0)


def _tcmid_body(maxB_ref, AB_ref, batch_ref, We_ref, be_ref,
                x_ref, AB2_ref):
    cnt = _count(batch_ref)

    def b0(ds, c):
        mb = maxB_ref[ds, :]
        x_ref[ds, :] = jnp.where(jnp.isfinite(mb), AB_ref[ds, _LO] + mb, 0.0)
        return c

    _chunk_loop(b0, 0)
    _norm_lrelu(batch_ref, cnt, x_ref, _ALL, H_C)
    We = We_ref[:]

    def b1(ds, c):
        xl = x_ref[ds, :]
        q = _dotp(xl, We[H_C:, :])
        AB2_ref[ds, _LO] = _dotp(xl, We[:H_C, :]) - q + be_ref[:]
        AB2_ref[ds, _HI] = q
        return c

    _chunk_loop(b1, 0)


def _tcx4_body(maxB_ref, AB_ref, batch_ref, x_ref):
    cnt = _count(batch_ref)

    def b0(ds, c):
        mb = maxB_ref[ds, :]
        x_ref[ds, :] = jnp.where(jnp.isfinite(mb), AB_ref[ds, _LO] + mb, 0.0)
        return c

    _chunk_loop(b0, 0)
    _norm_lrelu(batch_ref, cnt, x_ref, _ALL, H_C)


def _tcatt_body(xc_ref, batch_ref, Wg_ref, bg_ref, tail_ref):
    cnt = _count(batch_ref)

    def gate_at(ds):
        return _dotp(xc_ref[ds, :], Wg_ref[:]) + bg_ref[:]

    def b1(ds, gmax):
        g = gate_at(ds)
        m = _mask_at(batch_ref, ds)
        parts = []
        for gi in range(NUM_GRAPHS_C):
            sel = m[:, gi:gi + 1] > 0
            parts.append(jnp.max(jnp.where(sel, g, NEG_INF), axis=0,
                                 keepdims=True))
        return jnp.maximum(gmax, jnp.concatenate(parts, axis=0))

    gmax = _chunk_loop(b1, jnp.full((NUM_GRAPHS_C, 1), NEG_INF, jnp.float32))
    gmax = jnp.where(jnp.isfinite(gmax), gmax, 0.0)

    def b2(ds, denom):
        m = _mask_at(batch_ref, ds)
        e = jnp.exp(gate_at(ds) - _dot(m, gmax))
        return denom + _dot_t(m, e)

    denom = _chunk_loop(b2, jnp.zeros((NUM_GRAPHS_C, 1), jnp.float32))

    def b3(ds, jaw):
        m = _mask_at(batch_ref, ds)
        e = jnp.exp(gate_at(ds) - _dot(m, gmax))
        att = e / (_dot(m, denom) + 1e-16)
        return jaw + _dot_t(m, att * xc_ref[ds, :])

    jaw = _chunk_loop(b3, jnp.zeros((NUM_GRAPHS_C, 4 * H_C), jnp.float32))

    def b4(ds, c):
        tail_ref[ds, :] = _dot(_mask_at(batch_ref, ds), jaw)
        return c

    _chunk_loop(b4, 0)


def _f32(*shape):
    return jax.ShapeDtypeStruct(shape, jnp.float32)


def _params():
    return pltpu.CompilerParams(vmem_limit_bytes=100 * 1024 * 1024)


_tc1 = pl.pallas_call(
    _tc1_body,
    out_shape=[_f32(N_NODES_C, 2 * H_C)],
    scratch_shapes=[pltpu.VMEM((N_NODES_C, 2 * H_C), jnp.float32),
                    pltpu.VMEM((N_NODES_C, 128), jnp.float32),
                    pltpu.VMEM((N_NODES_C, 256), jnp.float32)],
    compiler_params=_params(),
)

_tcm = pl.pallas_call(
    _tcmid_body,
    out_shape=[_f32(N_NODES_C, H_C), _f32(N_NODES_C, 2 * H_C)],
    compiler_params=_params(),
)

_tcx4 = pl.pallas_call(
    _tcx4_body,
    out_shape=[_f32(N_NODES_C, H_C)],
    compiler_params=_params(),
)

_tcatt = pl.pallas_call(
    _tcatt_body,
    out_shape=[_f32(N_NODES_C, 4 * H_C)],
    compiler_params=_params(),
)


from jax.experimental.pallas import tpu_sc as plsc

N_EDGES_C = 320000
NW = 32
NPW = 313
TRASH = NPW
EW_BIN = 8000
FLUSH = 2048
CAP = 324096
LW = 2048
GRP = LW // 16
NBUF = 8

_sc_mesh = plsc.VectorSubcoreMesh(core_axis_name="c", subcore_axis_name="s")


def _wid():
    return lax.axis_index("s") * 2 + lax.axis_index("c")


@functools.partial(
    pl.kernel,
    out_type=[jax.ShapeDtypeStruct((NW * CAP,), jnp.int32),
              jax.ShapeDtypeStruct((NW * 16,), jnp.int32)],
    mesh=_sc_mesh,
    scratch_types=[pltpu.VMEM((EW_BIN,), jnp.int32),
                   pltpu.VMEM((EW_BIN,), jnp.int32),
                   pltpu.VMEM((FLUSH + 32,), jnp.int32),
                   pltpu.VMEM((16,), jnp.int32)],
    compiler_params=pltpu.CompilerParams(needs_layout_passes=False),
)
def _sc_bin(src_hbm, dst_hbm, lists_hbm, counts_hbm, srcw, dstw, stage, cbuf):
    w = _wid()
    lo = w * NPW
    lbase = w * CAP
    lane = lax.broadcasted_iota(jnp.int32, (16,), 0)

    def window(t, carry):
        base = t * EW_BIN
        pltpu.sync_copy(src_hbm.at[pl.ds(base, EW_BIN)], srcw)
        pltpu.sync_copy(dst_hbm.at[pl.ds(base, EW_BIN)], dstw)

        def group(k, car):
            off, flushes = car
            s = srcw[pl.ds(k * 16, 16)]
            d = dstw[pl.ds(k * 16, 16)]
            ld = d - lo
            sel = ld.astype(jnp.uint32) < jnp.uint32(NPW)
            p = (ld << 16) | s
            pre = jnp.where(sel, 1, 0)
            for sh in (1, 2, 4, 8):
                idxs = jnp.maximum(lane - sh, 0)
                pre = pre + jnp.where(lane >= sh, jnp.take(pre, idxs), 0)
            tgt = jnp.where(sel, off + pre - 1, FLUSH + 16)
            plsc.store_scatter(stage, [tgt], p)
            off = off + pre[15]

            def do_flush(a):
                o, f = a
                pltpu.sync_copy(
                    stage.at[pl.ds(0, FLUSH)],
                    lists_hbm.at[pl.ds(lbase + f * FLUSH, FLUSH)])
                stage[pl.ds(0, 16)] = stage[pl.ds(FLUSH, 16)]
                return o - FLUSH, f + 1

            return lax.cond(off >= FLUSH, do_flush, lambda a: a,
                            (off, flushes))

        return lax.fori_loop(0, EW_BIN // 16, group, carry)

    off, flushes = lax.fori_loop(0, N_EDGES_C // EW_BIN, window,
                                 (jnp.int32(0), jnp.int32(0)))
    pltpu.sync_copy(stage.at[pl.ds(0, FLUSH)],
                    lists_hbm.at[pl.ds(lbase + flushes * FLUSH, FLUSH)])
    cbuf[...] = (flushes * FLUSH + off) + jnp.zeros((16,), jnp.int32)
    pltpu.sync_copy(cbuf, counts_hbm.at[pl.ds(w * 16, 16)])


@functools.partial(
    pl.kernel,
    out_type=jax.ShapeDtypeStruct((NW * NPW * H_C,), jnp.float32),
    mesh=_sc_mesh,
    scratch_types=[pltpu.VMEM(((GRP + NBUF) * 16,), jnp.int32),
                   pltpu.VMEM(((NPW + 1) * H_C,), jnp.float32),
                   pltpu.VMEM((16,), jnp.int32)]
    + [pltpu.VMEM((16, H_C), jnp.float32)] * NBUF
    + [pltpu.VMEM((16,), jnp.int32)] * NBUF
    + [pltpu.SemaphoreType.DMA] * NBUF,
    compiler_params=pltpu.CompilerParams(needs_layout_passes=False,
                                         use_tc_tiling_on_sc=False),
)
def _sc_edge(B_hbm, lists_hbm, counts_hbm, out_hbm, listw, acc, cbuf, *scr):
    rows = list(scr[0:NBUF])
    ldbs = list(scr[NBUF:2 * NBUF])
    sems = list(scr[2 * NBUF:3 * NBUF])
    w = _wid()
    lbase = w * CAP
    iota = lax.broadcasted_iota(jnp.int32, (16,), 0)
    neg = jnp.full((16,), NEG_INF, jnp.float32)

    def init(k, c):
        acc[pl.ds(k * 16, 16)] = neg
        return c

    lax.fori_loop(0, (NPW + 1) * H_C // 16, init, 0)

    pltpu.sync_copy(counts_hbm.at[pl.ds(w * 16, 16)], cbuf)
    cnt = cbuf[pl.ds(0, 16)][0]

    def issue(b, g, t_base):
        gbase = g * 16
        p = listw[pl.ds(gbase, 16)]
        limit = jnp.where(gbase < LW, cnt, -1)
        valid = (t_base + gbase + iota) < limit
        s = jnp.where(valid, p & 0xFFFF, iota)
        ldbs[b][...] = jnp.where(valid, p >> 16, TRASH)
        pltpu.make_async_copy(B_hbm.at[s], rows[b], sems[b]).start()

    def process(b):
        pltpu.make_async_copy(B_hbm.at[iota], rows[b], sems[b]).wait()
        lv = ldbs[b][pl.ds(0, 16)] * H_C
        for e in range(16):
            base = lv[e]
            for cix in range(4):
                sl = pl.ds(base + cix * 16, 16)
                acc[sl] = jnp.maximum(acc[sl], rows[b][e, pl.ds(cix * 16, 16)])

    def window(t, c):
        t_base = t * LW
        pltpu.sync_copy(lists_hbm.at[pl.ds(lbase + t_base, LW)],
                        listw.at[pl.ds(0, LW)])
        for b in range(NBUF):
            issue(b, b, t_base)

        def chunk(cc, c2):
            for b in range(NBUF):
                process(b)
                issue(b, cc * NBUF + b + NBUF, t_base)
            return c2

        lax.fori_loop(0, GRP // NBUF, chunk, 0)
        for b in range(NBUF):
            process(b)
        return c

    lax.fori_loop(0, (cnt + LW - 1) // LW, window, 0)
    pltpu.sync_copy(acc.at[pl.ds(0, NPW * H_C)],
                    out_hbm.at[pl.ds(w * NPW * H_C, NPW * H_C)])


def kernel(x, edge_index, batch, W0, b0, Ws1, bs1, Ws2, bs2, Ws3, bs3,
           Wf1, bf1, Wf2, bf2, Wf3, bf3, We1, be1, We2, be2, We3, be3,
           We4, be4, Wg, bg):
    N = x.shape[0]
    batch2 = batch.reshape(N, 1).astype(jnp.int32)
    src = edge_index[0].astype(jnp.int32)
    dst = edge_index[1].astype(jnp.int32)

    def r2(b):
        return b.reshape(1, -1)

    lists, counts = _sc_bin(src, dst)

    def edge_max(AB):
        B = AB[:, H_C:]
        out = _sc_edge(B, lists, counts)
        return out.reshape(NW * NPW, H_C)[:N_NODES_C]

    (AB1,) = _tc1(x, batch2, W0, r2(b0), Ws1, r2(bs1), Ws2, r2(bs2),
                  Ws3, r2(bs3), Wf1, r2(bf1), Wf2, r2(bf2), Wf3, r2(bf3),
                  We1, r2(be1))

    m1 = edge_max(AB1)
    x1, AB2 = _tcm(m1, AB1, batch2, We2, r2(be2))
    m2 = edge_max(AB2)
    x2, AB3 = _tcm(m2, AB2, batch2, We3, r2(be3))
    m3 = edge_max(AB3)
    x3, AB4 = _tcm(m3, AB3, batch2, We4, r2(be4))
    m4 = edge_max(AB4)
    (x4,) = _tcx4(m4, AB4, batch2)

    xc = jnp.concatenate([x1, x2, x3, x4], axis=1)
    (tail,) = _tcatt(xc, batch2, Wg, r2(bg))
    return jnp.concatenate([xc, tail], axis=1)


# speedup vs baseline: 1.2086x; 1.2086x over previous
import functools
import jax
import jax.numpy as jnp
from jax import lax
from jax.experimental import pallas as pl
from jax.experimental.pallas import tpu as pltpu

N_NODES_C = 10000
NUM_GRAPHS_C = 8
H_C = 64
NEG_INF = float("-inf")
CH = 1000
NCH = N_NODES_C // CH

_HIGH = lax.Precision.HIGHEST


def _dot(a, b):
    return lax.dot_general(a, b, (((1,), (0,)), ((), ())),
                           precision=_HIGH, preferred_element_type=jnp.float32)


def _dotp(a, b):
    return lax.dot_general(a, b, (((1,), (0,)), ((), ())),
                           preferred_element_type=jnp.float32)


def _dot_t(a, b):
    return lax.dot_general(a, b, (((0,), (0,)), ((), ())),
                           precision=_HIGH, preferred_element_type=jnp.float32)


def _lrelu(v):
    return jnp.where(v > 0, v, 0.01 * v)


def _rows(i):
    return pl.ds(pl.multiple_of(i * CH, CH), CH)


def _chunk_loop(body, init):
    return lax.fori_loop(0, NCH, lambda i, c: body(_rows(i), c), init)


def _mask_at(batch_ref, ds):
    iota = lax.broadcasted_iota(jnp.int32, (CH, NUM_GRAPHS_C), 1)
    return (batch_ref[ds, :] == iota).astype(jnp.float32)


def _count(batch_ref):
    ones = jnp.ones((CH, 1), jnp.float32)

    def body(ds, cnt):
        return cnt + _dot_t(_mask_at(batch_ref, ds), ones)

    cnt = _chunk_loop(body, jnp.zeros((NUM_GRAPHS_C, 1), jnp.float32))
    return jnp.maximum(cnt, 1.0)


def _norm_lrelu(batch_ref, cnt, dst_ref, cols, width):
    def b1(ds, sums):
        return sums + _dot_t(_mask_at(batch_ref, ds), dst_ref[ds, cols])

    sums = _chunk_loop(b1, jnp.zeros((NUM_GRAPHS_C, width), jnp.float32))
    mean = sums / cnt

    def b2(ds, var):
        m = _mask_at(batch_ref, ds)
        hc = dst_ref[ds, cols] - _dot(m, mean)
        dst_ref[ds, cols] = hc
        return var + _dot_t(m, hc * hc)

    var = _chunk_loop(b2, jnp.zeros((NUM_GRAPHS_C, width), jnp.float32))
    rs = lax.rsqrt(var / cnt + 1e-5)

    def b3(ds, c):
        m = _mask_at(batch_ref, ds)
        dst_ref[ds, cols] = _lrelu(dst_ref[ds, cols] * _dot(m, rs))
        return c

    _chunk_loop(b3, 0)


def _lin_seq_ref(batch_ref, cnt, src_ref, src_cols, W, b, dst_ref, dst_cols,
                 width):
    def body(ds, c):
        dst_ref[ds, dst_cols] = _dotp(src_ref[ds, src_cols], W) + b
        return c

    _chunk_loop(body, 0)
    _norm_lrelu(batch_ref, cnt, dst_ref, dst_cols, width)


def _seg_max_cols(batch_ref, src_ref, cols, width):
    def body(ds, gm):
        m = _mask_at(batch_ref, ds)
        v = src_ref[ds, cols]
        parts = []
        for g in range(NUM_GRAPHS_C):
            sel = m[:, g:g + 1] > 0
            parts.append(jnp.max(jnp.where(sel, v, NEG_INF), axis=0,
                                 keepdims=True))
        return jnp.maximum(gm, jnp.concatenate(parts, axis=0))

    return _chunk_loop(
        body, jnp.full((NUM_GRAPHS_C, width), NEG_INF, jnp.float32))


_ALL = slice(None)
_LO = slice(0, H_C)
_HI = slice(H_C, 2 * H_C)


def _tc1_body(x_ref, batch_ref, W0_ref, b0_ref, Ws1_ref, bs1_ref, Ws2_ref,
              bs2_ref, Ws3_ref, bs3_ref, Wf1_ref, bf1_ref, Wf2_ref, bf2_ref,
              Wf3_ref, bf3_ref, We1_ref, be1_ref, AB_ref,
              x0t1_ref, t2_ref, t3_ref):
    cnt = _count(batch_ref)
    _lin_seq_ref(batch_ref, cnt, x_ref, _ALL, W0_ref[:], b0_ref[:],
                 x0t1_ref, _LO, H_C)
    _lin_seq_ref(batch_ref, cnt, x0t1_ref, _LO, Ws1_ref[:], bs1_ref[:],
                 x0t1_ref, _HI, 64)
    _lin_seq_ref(batch_ref, cnt, x0t1_ref, _HI, Ws2_ref[:], bs2_ref[:],
                 t2_ref, _ALL, 128)
    _lin_seq_ref(batch_ref, cnt, t2_ref, _ALL, Ws3_ref[:], bs3_ref[:],
                 t3_ref, _ALL, 256)
    tp = _seg_max_cols(batch_ref, t3_ref, _ALL, 256)
    tp = jnp.where(jnp.isfinite(tp), tp, 0.0)
    tp = _lrelu(_dotp(tp, Wf1_ref[:]) + bf1_ref[:])
    tp = _lrelu(_dotp(tp, Wf2_ref[:]) + bf2_ref[:])
    tp = _dotp(tp, Wf3_ref[:]) + bf3_ref[:]
    eye_flat = (lax.broadcasted_iota(jnp.int32, (1, H_C * H_C), 1) % (H_C + 1)
                == 0).astype(jnp.float32)
    trans = (tp + eye_flat).reshape(NUM_GRAPHS_C, H_C, H_C)
    We = We1_ref[:]

    def bT(ds, c):
        x0c = x0t1_ref[ds, _LO]
        m = _mask_at(batch_ref, ds)
        xT = jnp.zeros((CH, H_C), jnp.float32)
        for g in range(NUM_GRAPHS_C):
            xT = xT + m[:, g:g + 1] * _dotp(x0c, trans[g])
        q = _dotp(xT, We[H_C:, :])
        AB_ref[ds, _LO] = _dotp(xT, We[:H_C, :]) - q + be1_ref[:]
        AB_ref[ds, _HI] = q
        return c

    _chunk_loop(bT, 0)


def _tcmid_body(maxB_ref, AB_ref, batch_ref, We_ref, be_ref,
                x_ref, AB2_ref):
    cnt = _count(batch_ref)

    def b0(ds, c):
        mb = maxB_ref[ds, :]
        x_ref[ds, :] = jnp.where(jnp.isfinite(mb), AB_ref[ds, _LO] + mb, 0.0)
        return c

    _chunk_loop(b0, 0)
    _norm_lrelu(batch_ref, cnt, x_ref, _ALL, H_C)
    We = We_ref[:]

    def b1(ds, c):
        xl = x_ref[ds, :]
        q = _dotp(xl, We[H_C:, :])
        AB2_ref[ds, _LO] = _dotp(xl, We[:H_C, :]) - q + be_ref[:]
        AB2_ref[ds, _HI] = q
        return c

    _chunk_loop(b1, 0)


def _tcx4_body(maxB_ref, AB_ref, batch_ref, x_ref):
    cnt = _count(batch_ref)

    def b0(ds, c):
        mb = maxB_ref[ds, :]
        x_ref[ds, :] = jnp.where(jnp.isfinite(mb), AB_ref[ds, _LO] + mb, 0.0)
        return c

    _chunk_loop(b0, 0)
    _norm_lrelu(batch_ref, cnt, x_ref, _ALL, H_C)


def _tcatt_body(xc_ref, batch_ref, Wg_ref, bg_ref, tail_ref):
    cnt = _count(batch_ref)

    def gate_at(ds):
        return _dotp(xc_ref[ds, :], Wg_ref[:]) + bg_ref[:]

    def b1(ds, gmax):
        g = gate_at(ds)
        m = _mask_at(batch_ref, ds)
        parts = []
        for gi in range(NUM_GRAPHS_C):
            sel = m[:, gi:gi + 1] > 0
            parts.append(jnp.max(jnp.where(sel, g, NEG_INF), axis=0,
                                 keepdims=True))
        return jnp.maximum(gmax, jnp.concatenate(parts, axis=0))

    gmax = _chunk_loop(b1, jnp.full((NUM_GRAPHS_C, 1), NEG_INF, jnp.float32))
    gmax = jnp.where(jnp.isfinite(gmax), gmax, 0.0)

    def b2(ds, denom):
        m = _mask_at(batch_ref, ds)
        e = jnp.exp(gate_at(ds) - _dot(m, gmax))
        return denom + _dot_t(m, e)

    denom = _chunk_loop(b2, jnp.zeros((NUM_GRAPHS_C, 1), jnp.float32))

    def b3(ds, jaw):
        m = _mask_at(batch_ref, ds)
        e = jnp.exp(gate_at(ds) - _dot(m, gmax))
        att = e / (_dot(m, denom) + 1e-16)
        return jaw + _dot_t(m, att * xc_ref[ds, :])

    jaw = _chunk_loop(b3, jnp.zeros((NUM_GRAPHS_C, 4 * H_C), jnp.float32))

    def b4(ds, c):
        tail_ref[ds, :] = _dot(_mask_at(batch_ref, ds), jaw)
        return c

    _chunk_loop(b4, 0)


def _f32(*shape):
    return jax.ShapeDtypeStruct(shape, jnp.float32)


def _params():
    return pltpu.CompilerParams(vmem_limit_bytes=100 * 1024 * 1024)


_tc1 = pl.pallas_call(
    _tc1_body,
    out_shape=[_f32(N_NODES_C, 2 * H_C)],
    scratch_shapes=[pltpu.VMEM((N_NODES_C, 2 * H_C), jnp.float32),
                    pltpu.VMEM((N_NODES_C, 128), jnp.float32),
                    pltpu.VMEM((N_NODES_C, 256), jnp.float32)],
    compiler_params=_params(),
)

_tcm = pl.pallas_call(
    _tcmid_body,
    out_shape=[_f32(N_NODES_C, H_C), _f32(N_NODES_C, 2 * H_C)],
    compiler_params=_params(),
)

_tcx4 = pl.pallas_call(
    _tcx4_body,
    out_shape=[_f32(N_NODES_C, H_C)],
    compiler_params=_params(),
)

_tcatt = pl.pallas_call(
    _tcatt_body,
    out_shape=[_f32(N_NODES_C, 4 * H_C)],
    compiler_params=_params(),
)


from jax.experimental.pallas import tpu_sc as plsc

N_EDGES_C = 320000
NW = 32
NPW = 313
TRASH = NPW
EW_BIN = 8000
FLUSH = 2048
CAP = 324096
LW = 2048
GRP = LW // 16
NBUF = 8

_sc_mesh = plsc.VectorSubcoreMesh(core_axis_name="c", subcore_axis_name="s")


def _wid():
    return lax.axis_index("s") * 2 + lax.axis_index("c")


@functools.partial(
    pl.kernel,
    out_type=[jax.ShapeDtypeStruct((NW * CAP,), jnp.int32),
              jax.ShapeDtypeStruct((NW * 16,), jnp.int32)],
    mesh=_sc_mesh,
    scratch_types=[pltpu.VMEM((EW_BIN,), jnp.int32),
                   pltpu.VMEM((EW_BIN,), jnp.int32),
                   pltpu.VMEM((FLUSH + 32,), jnp.int32),
                   pltpu.VMEM((16,), jnp.int32)],
    compiler_params=pltpu.CompilerParams(needs_layout_passes=False),
)
def _sc_bin(src_hbm, dst_hbm, lists_hbm, counts_hbm, srcw, dstw, stage, cbuf):
    w = _wid()
    lo = w * NPW
    lbase = w * CAP
    lane = lax.broadcasted_iota(jnp.int32, (16,), 0)

    def window(t, carry):
        base = t * EW_BIN
        pltpu.sync_copy(src_hbm.at[pl.ds(base, EW_BIN)], srcw)
        pltpu.sync_copy(dst_hbm.at[pl.ds(base, EW_BIN)], dstw)

        def sort_group(k):
            s = srcw[pl.ds(k * 16, 16)]
            d = dstw[pl.ds(k * 16, 16)]
            ld = d - lo
            sel = ld.astype(jnp.uint32) < jnp.uint32(NPW)
            p = (ld << 16) | s
            key = jnp.where(sel, jnp.uint32(0), jnp.uint32(1))
            _, pv = plsc.sort_key_val(key, p)
            return pv, plsc.all_reduce_population_count(sel)[0]

        def group(k, car):
            off, flushes = car
            pv0, c0 = sort_group(2 * k)
            pv1, c1 = sort_group(2 * k + 1)
            stage[pl.ds(off, 16)] = pv0
            off1 = off + c0
            stage[pl.ds(off1, 16)] = pv1
            off = off1 + c1

            def do_flush(a):
                o, f = a
                pltpu.sync_copy(
                    stage.at[pl.ds(0, FLUSH)],
                    lists_hbm.at[pl.ds(lbase + f * FLUSH, FLUSH)])
                stage[pl.ds(0, 16)] = stage[pl.ds(FLUSH, 16)]
                stage[pl.ds(16, 16)] = stage[pl.ds(FLUSH + 16, 16)]
                return o - FLUSH, f + 1

            return lax.cond(off >= FLUSH, do_flush, lambda a: a,
                            (off, flushes))

        return lax.fori_loop(0, EW_BIN // 32, group, carry)

    off, flushes = lax.fori_loop(0, N_EDGES_C // EW_BIN, window,
                                 (jnp.int32(0), jnp.int32(0)))
    pltpu.sync_copy(stage.at[pl.ds(0, FLUSH)],
                    lists_hbm.at[pl.ds(lbase + flushes * FLUSH, FLUSH)])
    cbuf[...] = (flushes * FLUSH + off) + jnp.zeros((16,), jnp.int32)
    pltpu.sync_copy(cbuf, counts_hbm.at[pl.ds(w * 16, 16)])


@functools.partial(
    pl.kernel,
    out_type=jax.ShapeDtypeStruct((NW * NPW * H_C,), jnp.float32),
    mesh=_sc_mesh,
    scratch_types=[pltpu.VMEM(((GRP + NBUF) * 16,), jnp.int32),
                   pltpu.VMEM(((NPW + 1) * H_C,), jnp.float32),
                   pltpu.VMEM((16,), jnp.int32)]
    + [pltpu.VMEM((16, H_C), jnp.float32)] * NBUF
    + [pltpu.VMEM((16,), jnp.int32)] * NBUF
    + [pltpu.SemaphoreType.DMA] * NBUF,
    compiler_params=pltpu.CompilerParams(needs_layout_passes=False,
                                         use_tc_tiling_on_sc=False),
)
def _sc_edge(B_hbm, lists_hbm, counts_hbm, out_hbm, listw, acc, cbuf, *scr):
    rows = list(scr[0:NBUF])
    ldbs = list(scr[NBUF:2 * NBUF])
    sems = list(scr[2 * NBUF:3 * NBUF])
    w = _wid()
    lbase = w * CAP
    iota = lax.broadcasted_iota(jnp.int32, (16,), 0)
    neg = jnp.full((16,), NEG_INF, jnp.float32)

    def init(k, c):
        acc[pl.ds(k * 16, 16)] = neg
        return c

    lax.fori_loop(0, (NPW + 1) * H_C // 16, init, 0)

    pltpu.sync_copy(counts_hbm.at[pl.ds(w * 16, 16)], cbuf)
    cnt = cbuf[pl.ds(0, 16)][0]

    def issue(b, g, t_base):
        gbase = g * 16
        p = listw[pl.ds(gbase, 16)]
        limit = jnp.where(gbase < LW, cnt, -1)
        valid = (t_base + gbase + iota) < limit
        s = jnp.where(valid, p & 0xFFFF, iota)
        ldbs[b][...] = jnp.where(valid, p >> 16, TRASH)
        pltpu.make_async_copy(B_hbm.at[s], rows[b], sems[b]).start()

    def process(b):
        pltpu.make_async_copy(B_hbm.at[iota], rows[b], sems[b]).wait()
        lv = ldbs[b][pl.ds(0, 16)] * H_C
        for e in range(16):
            base = lv[e]
            for cix in range(4):
                sl = pl.ds(base + cix * 16, 16)
                acc[sl] = jnp.maximum(acc[sl], rows[b][e, pl.ds(cix * 16, 16)])

    def window(t, c):
        t_base = t * LW
        pltpu.sync_copy(lists_hbm.at[pl.ds(lbase + t_base, LW)],
                        listw.at[pl.ds(0, LW)])
        for b in range(NBUF):
            issue(b, b, t_base)

        def chunk(cc, c2):
            for b in range(NBUF):
                process(b)
                issue(b, cc * NBUF + b + NBUF, t_base)
            return c2

        lax.fori_loop(0, GRP // NBUF, chunk, 0)
        for b in range(NBUF):
            process(b)
        return c

    lax.fori_loop(0, (cnt + LW - 1) // LW, window, 0)
    pltpu.sync_copy(acc.at[pl.ds(0, NPW * H_C)],
                    out_hbm.at[pl.ds(w * NPW * H_C, NPW * H_C)])


def kernel(x, edge_index, batch, W0, b0, Ws1, bs1, Ws2, bs2, Ws3, bs3,
           Wf1, bf1, Wf2, bf2, Wf3, bf3, We1, be1, We2, be2, We3, be3,
           We4, be4, Wg, bg):
    N = x.shape[0]
    batch2 = batch.reshape(N, 1).astype(jnp.int32)
    src = edge_index[0].astype(jnp.int32)
    dst = edge_index[1].astype(jnp.int32)

    def r2(b):
        return b.reshape(1, -1)

    lists, counts = _sc_bin(src, dst)

    def edge_max(AB):
        B = AB[:, H_C:]
        out = _sc_edge(B, lists, counts)
        return out.reshape(NW * NPW, H_C)[:N_NODES_C]

    (AB1,) = _tc1(x, batch2, W0, r2(b0), Ws1, r2(bs1), Ws2, r2(bs2),
                  Ws3, r2(bs3), Wf1, r2(bf1), Wf2, r2(bf2), Wf3, r2(bf3),
                  We1, r2(be1))

    m1 = edge_max(AB1)
    x1, AB2 = _tcm(m1, AB1, batch2, We2, r2(be2))
    m2 = edge_max(AB2)
    x2, AB3 = _tcm(m2, AB2, batch2, We3, r2(be3))
    m3 = edge_max(AB3)
    x3, AB4 = _tcm(m3, AB3, batch2, We4, r2(be4))
    m4 = edge_max(AB4)
    (x4,) = _tcx4(m4, AB4, batch2)

    xc = jnp.concatenate([x1, x2, x3, x4], axis=1)
    (tail,) = _tcatt(xc, batch2, Wg, r2(bg))
    return jnp.concatenate([xc, tail], axis=1)


# speedup vs baseline: 1.4208x; 1.1756x over previous
import functools
import jax
import jax.numpy as jnp
from jax import lax
from jax.experimental import pallas as pl
from jax.experimental.pallas import tpu as pltpu

N_NODES_C = 10000
NUM_GRAPHS_C = 8
H_C = 64
NEG_INF = float("-inf")
CH = 1000
NCH = N_NODES_C // CH

_HIGH = lax.Precision.HIGHEST


def _dot(a, b):
    return lax.dot_general(a, b, (((1,), (0,)), ((), ())),
                           precision=_HIGH, preferred_element_type=jnp.float32)


def _dotp(a, b):
    return lax.dot_general(a, b, (((1,), (0,)), ((), ())),
                           preferred_element_type=jnp.float32)


def _dot_t(a, b):
    return lax.dot_general(a, b, (((0,), (0,)), ((), ())),
                           precision=_HIGH, preferred_element_type=jnp.float32)


def _lrelu(v):
    return jnp.where(v > 0, v, 0.01 * v)


def _rows(i):
    return pl.ds(pl.multiple_of(i * CH, CH), CH)


def _chunk_loop(body, init):
    return lax.fori_loop(0, NCH, lambda i, c: body(_rows(i), c), init)


def _mask_at(batch_ref, ds):
    iota = lax.broadcasted_iota(jnp.int32, (CH, NUM_GRAPHS_C), 1)
    return (batch_ref[ds, :] == iota).astype(jnp.float32)


def _count(batch_ref):
    ones = jnp.ones((CH, 1), jnp.float32)

    def body(ds, cnt):
        return cnt + _dot_t(_mask_at(batch_ref, ds), ones)

    cnt = _chunk_loop(body, jnp.zeros((NUM_GRAPHS_C, 1), jnp.float32))
    return jnp.maximum(cnt, 1.0)


def _norm_lrelu(batch_ref, cnt, dst_ref, cols, width):
    def b1(ds, sums):
        return sums + _dot_t(_mask_at(batch_ref, ds), dst_ref[ds, cols])

    sums = _chunk_loop(b1, jnp.zeros((NUM_GRAPHS_C, width), jnp.float32))
    mean = sums / cnt

    def b2(ds, var):
        m = _mask_at(batch_ref, ds)
        hc = dst_ref[ds, cols] - _dot(m, mean)
        dst_ref[ds, cols] = hc
        return var + _dot_t(m, hc * hc)

    var = _chunk_loop(b2, jnp.zeros((NUM_GRAPHS_C, width), jnp.float32))
    rs = lax.rsqrt(var / cnt + 1e-5)

    def b3(ds, c):
        m = _mask_at(batch_ref, ds)
        dst_ref[ds, cols] = _lrelu(dst_ref[ds, cols] * _dot(m, rs))
        return c

    _chunk_loop(b3, 0)


def _lin_seq_ref(batch_ref, cnt, src_ref, src_cols, W, b, dst_ref, dst_cols,
                 width):
    def body(ds, c):
        dst_ref[ds, dst_cols] = _dotp(src_ref[ds, src_cols], W) + b
        return c

    _chunk_loop(body, 0)
    _norm_lrelu(batch_ref, cnt, dst_ref, dst_cols, width)


def _seg_max_cols(batch_ref, src_ref, cols, width):
    def body(ds, gm):
        m = _mask_at(batch_ref, ds)
        v = src_ref[ds, cols]
        parts = []
        for g in range(NUM_GRAPHS_C):
            sel = m[:, g:g + 1] > 0
            parts.append(jnp.max(jnp.where(sel, v, NEG_INF), axis=0,
                                 keepdims=True))
        return jnp.maximum(gm, jnp.concatenate(parts, axis=0))

    return _chunk_loop(
        body, jnp.full((NUM_GRAPHS_C, width), NEG_INF, jnp.float32))


_ALL = slice(None)
_LO = slice(0, H_C)
_HI = slice(H_C, 2 * H_C)


def _tc1_body(x_ref, batch_ref, W0_ref, b0_ref, Ws1_ref, bs1_ref, Ws2_ref,
              bs2_ref, Ws3_ref, bs3_ref, Wf1_ref, bf1_ref, Wf2_ref, bf2_ref,
              Wf3_ref, bf3_ref, We1_ref, be1_ref, AB_ref,
              x0t1_ref, t2_ref, t3_ref):
    cnt = _count(batch_ref)
    _lin_seq_ref(batch_ref, cnt, x_ref, _ALL, W0_ref[:], b0_ref[:],
                 x0t1_ref, _LO, H_C)
    _lin_seq_ref(batch_ref, cnt, x0t1_ref, _LO, Ws1_ref[:], bs1_ref[:],
                 x0t1_ref, _HI, 64)
    _lin_seq_ref(batch_ref, cnt, x0t1_ref, _HI, Ws2_ref[:], bs2_ref[:],
                 t2_ref, _ALL, 128)
    _lin_seq_ref(batch_ref, cnt, t2_ref, _ALL, Ws3_ref[:], bs3_ref[:],
                 t3_ref, _ALL, 256)
    tp = _seg_max_cols(batch_ref, t3_ref, _ALL, 256)
    tp = jnp.where(jnp.isfinite(tp), tp, 0.0)
    tp = _lrelu(_dotp(tp, Wf1_ref[:]) + bf1_ref[:])
    tp = _lrelu(_dotp(tp, Wf2_ref[:]) + bf2_ref[:])
    tp = _dotp(tp, Wf3_ref[:]) + bf3_ref[:]
    eye_flat = (lax.broadcasted_iota(jnp.int32, (1, H_C * H_C), 1) % (H_C + 1)
                == 0).astype(jnp.float32)
    trans = (tp + eye_flat).reshape(NUM_GRAPHS_C, H_C, H_C)
    We = We1_ref[:]

    def bT(ds, c):
        x0c = x0t1_ref[ds, _LO]
        m = _mask_at(batch_ref, ds)
        xT = jnp.zeros((CH, H_C), jnp.float32)
        for g in range(NUM_GRAPHS_C):
            xT = xT + m[:, g:g + 1] * _dotp(x0c, trans[g])
        q = _dotp(xT, We[H_C:, :])
        AB_ref[ds, _LO] = _dotp(xT, We[:H_C, :]) - q + be1_ref[:]
        AB_ref[ds, _HI] = q
        return c

    _chunk_loop(bT, 0)


def _tcmid_body(maxB_ref, AB_ref, batch_ref, We_ref, be_ref,
                x_ref, AB2_ref):
    cnt = _count(batch_ref)

    def b0(ds, c):
        mb = maxB_ref[ds, :]
        x_ref[ds, :] = jnp.where(jnp.isfinite(mb), AB_ref[ds, _LO] + mb, 0.0)
        return c

    _chunk_loop(b0, 0)
    _norm_lrelu(batch_ref, cnt, x_ref, _ALL, H_C)
    We = We_ref[:]

    def b1(ds, c):
        xl = x_ref[ds, :]
        q = _dotp(xl, We[H_C:, :])
        AB2_ref[ds, _LO] = _dotp(xl, We[:H_C, :]) - q + be_ref[:]
        AB2_ref[ds, _HI] = q
        return c

    _chunk_loop(b1, 0)


def _tcx4_body(maxB_ref, AB_ref, batch_ref, x_ref):
    cnt = _count(batch_ref)

    def b0(ds, c):
        mb = maxB_ref[ds, :]
        x_ref[ds, :] = jnp.where(jnp.isfinite(mb), AB_ref[ds, _LO] + mb, 0.0)
        return c

    _chunk_loop(b0, 0)
    _norm_lrelu(batch_ref, cnt, x_ref, _ALL, H_C)


def _tcatt_body(xc_ref, batch_ref, Wg_ref, bg_ref, tail_ref):
    cnt = _count(batch_ref)

    def gate_at(ds):
        return _dotp(xc_ref[ds, :], Wg_ref[:]) + bg_ref[:]

    def b1(ds, gmax):
        g = gate_at(ds)
        m = _mask_at(batch_ref, ds)
        parts = []
        for gi in range(NUM_GRAPHS_C):
            sel = m[:, gi:gi + 1] > 0
            parts.append(jnp.max(jnp.where(sel, g, NEG_INF), axis=0,
                                 keepdims=True))
        return jnp.maximum(gmax, jnp.concatenate(parts, axis=0))

    gmax = _chunk_loop(b1, jnp.full((NUM_GRAPHS_C, 1), NEG_INF, jnp.float32))
    gmax = jnp.where(jnp.isfinite(gmax), gmax, 0.0)

    def b2(ds, denom):
        m = _mask_at(batch_ref, ds)
        e = jnp.exp(gate_at(ds) - _dot(m, gmax))
        return denom + _dot_t(m, e)

    denom = _chunk_loop(b2, jnp.zeros((NUM_GRAPHS_C, 1), jnp.float32))

    def b3(ds, jaw):
        m = _mask_at(batch_ref, ds)
        e = jnp.exp(gate_at(ds) - _dot(m, gmax))
        att = e / (_dot(m, denom) + 1e-16)
        return jaw + _dot_t(m, att * xc_ref[ds, :])

    jaw = _chunk_loop(b3, jnp.zeros((NUM_GRAPHS_C, 4 * H_C), jnp.float32))

    def b4(ds, c):
        tail_ref[ds, :] = _dot(_mask_at(batch_ref, ds), jaw)
        return c

    _chunk_loop(b4, 0)


def _f32(*shape):
    return jax.ShapeDtypeStruct(shape, jnp.float32)


def _params():
    return pltpu.CompilerParams(vmem_limit_bytes=100 * 1024 * 1024)


_tc1 = pl.pallas_call(
    _tc1_body,
    out_shape=[_f32(N_NODES_C, 2 * H_C)],
    scratch_shapes=[pltpu.VMEM((N_NODES_C, 2 * H_C), jnp.float32),
                    pltpu.VMEM((N_NODES_C, 128), jnp.float32),
                    pltpu.VMEM((N_NODES_C, 256), jnp.float32)],
    compiler_params=_params(),
)

_tcm = pl.pallas_call(
    _tcmid_body,
    out_shape=[_f32(N_NODES_C, H_C), _f32(N_NODES_C, 2 * H_C)],
    compiler_params=_params(),
)

_tcx4 = pl.pallas_call(
    _tcx4_body,
    out_shape=[_f32(N_NODES_C, H_C)],
    compiler_params=_params(),
)

_tcatt = pl.pallas_call(
    _tcatt_body,
    out_shape=[_f32(N_NODES_C, 4 * H_C)],
    compiler_params=_params(),
)


from jax.experimental.pallas import tpu_sc as plsc

N_EDGES_C = 320000
NW = 32
NPW = 313
TRASH = NPW
EW_BIN = 8000
FLUSH = 2048
CAP = 324096
LW = 2048
GRP = LW // 16
NBUF = 4

_sc_mesh = plsc.VectorSubcoreMesh(core_axis_name="c", subcore_axis_name="s")


def _wid():
    return lax.axis_index("s") * 2 + lax.axis_index("c")


@functools.partial(
    pl.kernel,
    out_type=[jax.ShapeDtypeStruct((NW * CAP,), jnp.int32),
              jax.ShapeDtypeStruct((NW * 16,), jnp.int32)],
    mesh=_sc_mesh,
    scratch_types=[pltpu.VMEM((EW_BIN,), jnp.int32),
                   pltpu.VMEM((EW_BIN,), jnp.int32),
                   pltpu.VMEM((FLUSH + 32,), jnp.int32),
                   pltpu.VMEM((16,), jnp.int32)],
    compiler_params=pltpu.CompilerParams(needs_layout_passes=False),
)
def _sc_bin(src_hbm, dst_hbm, lists_hbm, counts_hbm, srcw, dstw, stage, cbuf):
    w = _wid()
    lo = w * NPW
    lbase = w * CAP
    lane = lax.broadcasted_iota(jnp.int32, (16,), 0)

    def window(t, carry):
        base = t * EW_BIN
        pltpu.sync_copy(src_hbm.at[pl.ds(base, EW_BIN)], srcw)
        pltpu.sync_copy(dst_hbm.at[pl.ds(base, EW_BIN)], dstw)

        def sort_group(k):
            s = srcw[pl.ds(k * 16, 16)]
            d = dstw[pl.ds(k * 16, 16)]
            ld = d - lo
            sel = ld.astype(jnp.uint32) < jnp.uint32(NPW)
            p = (ld << 16) | s
            key = jnp.where(sel, jnp.uint32(0), jnp.uint32(1))
            _, pv = plsc.sort_key_val(key, p)
            return pv, plsc.all_reduce_population_count(sel)[0]

        def group(k, car):
            off, flushes = car
            pv0, c0 = sort_group(2 * k)
            pv1, c1 = sort_group(2 * k + 1)
            stage[pl.ds(off, 16)] = pv0
            off1 = off + c0
            stage[pl.ds(off1, 16)] = pv1
            off = off1 + c1

            def do_flush(a):
                o, f = a
                pltpu.sync_copy(
                    stage.at[pl.ds(0, FLUSH)],
                    lists_hbm.at[pl.ds(lbase + f * FLUSH, FLUSH)])
                stage[pl.ds(0, 16)] = stage[pl.ds(FLUSH, 16)]
                stage[pl.ds(16, 16)] = stage[pl.ds(FLUSH + 16, 16)]
                return o - FLUSH, f + 1

            return lax.cond(off >= FLUSH, do_flush, lambda a: a,
                            (off, flushes))

        return lax.fori_loop(0, EW_BIN // 32, group, carry)

    off, flushes = lax.fori_loop(0, N_EDGES_C // EW_BIN, window,
                                 (jnp.int32(0), jnp.int32(0)))
    pltpu.sync_copy(stage.at[pl.ds(0, FLUSH)],
                    lists_hbm.at[pl.ds(lbase + flushes * FLUSH, FLUSH)])
    cbuf[...] = (flushes * FLUSH + off) + jnp.zeros((16,), jnp.int32)
    pltpu.sync_copy(cbuf, counts_hbm.at[pl.ds(w * 16, 16)])


@functools.partial(
    pl.kernel,
    out_type=jax.ShapeDtypeStruct((NW * NPW * H_C,), jnp.float32),
    mesh=_sc_mesh,
    scratch_types=[pltpu.VMEM(((GRP + NBUF) * 16,), jnp.int32),
                   pltpu.VMEM(((NPW + 1) * H_C,), jnp.float32),
                   pltpu.VMEM(((NPW + 1) * H_C,), jnp.float32),
                   pltpu.VMEM((16,), jnp.int32)]
    + [pltpu.VMEM((16, H_C), jnp.float32)] * NBUF
    + [pltpu.VMEM((16,), jnp.int32)] * NBUF
    + [pltpu.SemaphoreType.DMA] * NBUF,
    compiler_params=pltpu.CompilerParams(needs_layout_passes=False,
                                         use_tc_tiling_on_sc=False),
)
def _sc_edge(B_hbm, lists_hbm, counts_hbm, out_hbm, listw, acc, acc2, cbuf,
             *scr):
    rows = list(scr[0:NBUF])
    ldbs = list(scr[NBUF:2 * NBUF])
    sems = list(scr[2 * NBUF:3 * NBUF])
    w = _wid()
    lbase = w * CAP
    iota = lax.broadcasted_iota(jnp.int32, (16,), 0)
    neg = jnp.full((16,), NEG_INF, jnp.float32)

    def init(k, c):
        acc[pl.ds(k * 16, 16)] = neg
        acc2[pl.ds(k * 16, 16)] = neg
        return c

    lax.fori_loop(0, (NPW + 1) * H_C // 16, init, 0)

    pltpu.sync_copy(counts_hbm.at[pl.ds(w * 16, 16)], cbuf)
    cnt = cbuf[pl.ds(0, 16)][0]

    def issue(b, g, t_base):
        gbase = g * 16
        p = listw[pl.ds(gbase, 16)]
        limit = jnp.where(gbase < LW, cnt, -1)
        valid = (t_base + gbase + iota) < limit
        s = jnp.where(valid, p & 0xFFFF, iota)
        ldbs[b][...] = jnp.where(valid, p >> 16, TRASH)
        pltpu.make_async_copy(B_hbm.at[s], rows[b], sems[b]).start()

    def process(b):
        pltpu.make_async_copy(B_hbm.at[iota], rows[b], sems[b]).wait()
        lv = ldbs[b][pl.ds(0, 16)] * H_C
        for e in range(16):
            base = lv[e]
            tgt = acc if e % 2 == 0 else acc2
            for cix in range(4):
                sl = pl.ds(base + cix * 16, 16)
                tgt[sl] = jnp.maximum(tgt[sl], rows[b][e, pl.ds(cix * 16, 16)])

    def window(t, c):
        t_base = t * LW
        pltpu.sync_copy(lists_hbm.at[pl.ds(lbase + t_base, LW)],
                        listw.at[pl.ds(0, LW)])
        for b in range(NBUF):
            issue(b, b, t_base)

        def chunk(cc, c2):
            for b in range(NBUF):
                process(b)
                issue(b, cc * NBUF + b + NBUF, t_base)
            return c2

        lax.fori_loop(0, GRP // NBUF, chunk, 0)
        for b in range(NBUF):
            process(b)
        return c

    lax.fori_loop(0, (cnt + LW - 1) // LW, window, 0)

    def merge(k, c):
        sl = pl.ds(k * 16, 16)
        acc[sl] = jnp.maximum(acc[sl], acc2[sl])
        return c

    lax.fori_loop(0, NPW * H_C // 16, merge, 0)
    pltpu.sync_copy(acc.at[pl.ds(0, NPW * H_C)],
                    out_hbm.at[pl.ds(w * NPW * H_C, NPW * H_C)])


def kernel(x, edge_index, batch, W0, b0, Ws1, bs1, Ws2, bs2, Ws3, bs3,
           Wf1, bf1, Wf2, bf2, Wf3, bf3, We1, be1, We2, be2, We3, be3,
           We4, be4, Wg, bg):
    N = x.shape[0]
    batch2 = batch.reshape(N, 1).astype(jnp.int32)
    src = edge_index[0].astype(jnp.int32)
    dst = edge_index[1].astype(jnp.int32)

    def r2(b):
        return b.reshape(1, -1)

    lists, counts = _sc_bin(src, dst)

    def edge_max(AB):
        B = AB[:, H_C:]
        out = _sc_edge(B, lists, counts)
        return out.reshape(NW * NPW, H_C)[:N_NODES_C]

    (AB1,) = _tc1(x, batch2, W0, r2(b0), Ws1, r2(bs1), Ws2, r2(bs2),
                  Ws3, r2(bs3), Wf1, r2(bf1), Wf2, r2(bf2), Wf3, r2(bf3),
                  We1, r2(be1))

    m1 = edge_max(AB1)
    x1, AB2 = _tcm(m1, AB1, batch2, We2, r2(be2))
    m2 = edge_max(AB2)
    x2, AB3 = _tcm(m2, AB2, batch2, We3, r2(be3))
    m3 = edge_max(AB3)
    x3, AB4 = _tcm(m3, AB3, batch2, We4, r2(be4))
    m4 = edge_max(AB4)
    (x4,) = _tcx4(m4, AB4, batch2)

    xc = jnp.concatenate([x1, x2, x3, x4], axis=1)
    (tail,) = _tcatt(xc, batch2, Wg, r2(bg))
    return jnp.concatenate([xc, tail], axis=1)


# speedup vs baseline: 1.4230x; 1.0015x over previous
import functools
import jax
import jax.numpy as jnp
from jax import lax
from jax.experimental import pallas as pl
from jax.experimental.pallas import tpu as pltpu

N_NODES_C = 10000
NUM_GRAPHS_C = 8
H_C = 64
NEG_INF = float("-inf")
CH = 1000
NCH = N_NODES_C // CH

_HIGH = lax.Precision.HIGHEST


def _dot(a, b):
    return lax.dot_general(a, b, (((1,), (0,)), ((), ())),
                           precision=_HIGH, preferred_element_type=jnp.float32)


def _dotp(a, b):
    return lax.dot_general(a, b, (((1,), (0,)), ((), ())),
                           preferred_element_type=jnp.float32)


def _dot_t(a, b):
    return lax.dot_general(a, b, (((0,), (0,)), ((), ())),
                           precision=_HIGH, preferred_element_type=jnp.float32)


def _lrelu(v):
    return jnp.where(v > 0, v, 0.01 * v)


def _rows(i):
    return pl.ds(pl.multiple_of(i * CH, CH), CH)


def _chunk_loop(body, init):
    return lax.fori_loop(0, NCH, lambda i, c: body(_rows(i), c), init)


def _mask_at(batch_ref, ds):
    iota = lax.broadcasted_iota(jnp.int32, (CH, NUM_GRAPHS_C), 1)
    return (batch_ref[ds, :] == iota).astype(jnp.float32)


def _count(batch_ref):
    ones = jnp.ones((CH, 1), jnp.float32)

    def body(ds, cnt):
        return cnt + _dot_t(_mask_at(batch_ref, ds), ones)

    cnt = _chunk_loop(body, jnp.zeros((NUM_GRAPHS_C, 1), jnp.float32))
    return jnp.maximum(cnt, 1.0)


def _norm_lrelu(batch_ref, cnt, dst_ref, cols, width):
    def b1(ds, sums):
        return sums + _dot_t(_mask_at(batch_ref, ds), dst_ref[ds, cols])

    sums = _chunk_loop(b1, jnp.zeros((NUM_GRAPHS_C, width), jnp.float32))
    mean = sums / cnt

    def b2(ds, var):
        m = _mask_at(batch_ref, ds)
        hc = dst_ref[ds, cols] - _dot(m, mean)
        dst_ref[ds, cols] = hc
        return var + _dot_t(m, hc * hc)

    var = _chunk_loop(b2, jnp.zeros((NUM_GRAPHS_C, width), jnp.float32))
    rs = lax.rsqrt(var / cnt + 1e-5)

    def b3(ds, c):
        m = _mask_at(batch_ref, ds)
        dst_ref[ds, cols] = _lrelu(dst_ref[ds, cols] * _dot(m, rs))
        return c

    _chunk_loop(b3, 0)


def _lin_seq_ref(batch_ref, cnt, src_ref, src_cols, W, b, dst_ref, dst_cols,
                 width):
    def body(ds, c):
        dst_ref[ds, dst_cols] = _dotp(src_ref[ds, src_cols], W) + b
        return c

    _chunk_loop(body, 0)
    _norm_lrelu(batch_ref, cnt, dst_ref, dst_cols, width)


def _seg_max_cols(batch_ref, src_ref, cols, width):
    def body(ds, gm):
        m = _mask_at(batch_ref, ds)
        v = src_ref[ds, cols]
        parts = []
        for g in range(NUM_GRAPHS_C):
            sel = m[:, g:g + 1] > 0
            parts.append(jnp.max(jnp.where(sel, v, NEG_INF), axis=0,
                                 keepdims=True))
        return jnp.maximum(gm, jnp.concatenate(parts, axis=0))

    return _chunk_loop(
        body, jnp.full((NUM_GRAPHS_C, width), NEG_INF, jnp.float32))


_ALL = slice(None)
_LO = slice(0, H_C)
_HI = slice(H_C, 2 * H_C)


def _tc1_body(x_ref, batch_ref, W0_ref, b0_ref, Ws1_ref, bs1_ref, Ws2_ref,
              bs2_ref, Ws3_ref, bs3_ref, Wf1_ref, bf1_ref, Wf2_ref, bf2_ref,
              Wf3_ref, bf3_ref, We1_ref, be1_ref, AB_ref,
              x0t1_ref, t2_ref, t3_ref):
    cnt = _count(batch_ref)
    _lin_seq_ref(batch_ref, cnt, x_ref, _ALL, W0_ref[:], b0_ref[:],
                 x0t1_ref, _LO, H_C)
    _lin_seq_ref(batch_ref, cnt, x0t1_ref, _LO, Ws1_ref[:], bs1_ref[:],
                 x0t1_ref, _HI, 64)
    _lin_seq_ref(batch_ref, cnt, x0t1_ref, _HI, Ws2_ref[:], bs2_ref[:],
                 t2_ref, _ALL, 128)
    _lin_seq_ref(batch_ref, cnt, t2_ref, _ALL, Ws3_ref[:], bs3_ref[:],
                 t3_ref, _ALL, 256)
    tp = _seg_max_cols(batch_ref, t3_ref, _ALL, 256)
    tp = jnp.where(jnp.isfinite(tp), tp, 0.0)
    tp = _lrelu(_dotp(tp, Wf1_ref[:]) + bf1_ref[:])
    tp = _lrelu(_dotp(tp, Wf2_ref[:]) + bf2_ref[:])
    tp = _dotp(tp, Wf3_ref[:]) + bf3_ref[:]
    eye_flat = (lax.broadcasted_iota(jnp.int32, (1, H_C * H_C), 1) % (H_C + 1)
                == 0).astype(jnp.float32)
    trans = (tp + eye_flat).reshape(NUM_GRAPHS_C, H_C, H_C)
    We = We1_ref[:]

    def bT(ds, c):
        x0c = x0t1_ref[ds, _LO]
        m = _mask_at(batch_ref, ds)
        xT = jnp.zeros((CH, H_C), jnp.float32)
        for g in range(NUM_GRAPHS_C):
            xT = xT + m[:, g:g + 1] * _dotp(x0c, trans[g])
        q = _dotp(xT, We[H_C:, :])
        AB_ref[ds, _LO] = _dotp(xT, We[:H_C, :]) - q + be1_ref[:]
        AB_ref[ds, _HI] = q
        return c

    _chunk_loop(bT, 0)


def _tcmid_body(maxB_ref, AB_ref, batch_ref, We_ref, be_ref,
                x_ref, AB2_ref):
    cnt = _count(batch_ref)

    def b0(ds, c):
        mb = maxB_ref[ds, :]
        x_ref[ds, :] = jnp.where(jnp.isfinite(mb), AB_ref[ds, _LO] + mb, 0.0)
        return c

    _chunk_loop(b0, 0)
    _norm_lrelu(batch_ref, cnt, x_ref, _ALL, H_C)
    We = We_ref[:]

    def b1(ds, c):
        xl = x_ref[ds, :]
        q = _dotp(xl, We[H_C:, :])
        AB2_ref[ds, _LO] = _dotp(xl, We[:H_C, :]) - q + be_ref[:]
        AB2_ref[ds, _HI] = q
        return c

    _chunk_loop(b1, 0)


def _tcx4_body(maxB_ref, AB_ref, batch_ref, x_ref):
    cnt = _count(batch_ref)

    def b0(ds, c):
        mb = maxB_ref[ds, :]
        x_ref[ds, :] = jnp.where(jnp.isfinite(mb), AB_ref[ds, _LO] + mb, 0.0)
        return c

    _chunk_loop(b0, 0)
    _norm_lrelu(batch_ref, cnt, x_ref, _ALL, H_C)


def _tcatt_body(xc_ref, batch_ref, Wg_ref, bg_ref, tail_ref):
    cnt = _count(batch_ref)

    def gate_at(ds):
        return _dotp(xc_ref[ds, :], Wg_ref[:]) + bg_ref[:]

    def b1(ds, gmax):
        g = gate_at(ds)
        m = _mask_at(batch_ref, ds)
        parts = []
        for gi in range(NUM_GRAPHS_C):
            sel = m[:, gi:gi + 1] > 0
            parts.append(jnp.max(jnp.where(sel, g, NEG_INF), axis=0,
                                 keepdims=True))
        return jnp.maximum(gmax, jnp.concatenate(parts, axis=0))

    gmax = _chunk_loop(b1, jnp.full((NUM_GRAPHS_C, 1), NEG_INF, jnp.float32))
    gmax = jnp.where(jnp.isfinite(gmax), gmax, 0.0)

    def b2(ds, denom):
        m = _mask_at(batch_ref, ds)
        e = jnp.exp(gate_at(ds) - _dot(m, gmax))
        return denom + _dot_t(m, e)

    denom = _chunk_loop(b2, jnp.zeros((NUM_GRAPHS_C, 1), jnp.float32))

    def b3(ds, jaw):
        m = _mask_at(batch_ref, ds)
        e = jnp.exp(gate_at(ds) - _dot(m, gmax))
        att = e / (_dot(m, denom) + 1e-16)
        return jaw + _dot_t(m, att * xc_ref[ds, :])

    jaw = _chunk_loop(b3, jnp.zeros((NUM_GRAPHS_C, 4 * H_C), jnp.float32))

    def b4(ds, c):
        tail_ref[ds, :] = _dot(_mask_at(batch_ref, ds), jaw)
        return c

    _chunk_loop(b4, 0)


def _f32(*shape):
    return jax.ShapeDtypeStruct(shape, jnp.float32)


def _params():
    return pltpu.CompilerParams(vmem_limit_bytes=100 * 1024 * 1024)


_tc1 = pl.pallas_call(
    _tc1_body,
    out_shape=[_f32(N_NODES_C, 2 * H_C)],
    scratch_shapes=[pltpu.VMEM((N_NODES_C, 2 * H_C), jnp.float32),
                    pltpu.VMEM((N_NODES_C, 128), jnp.float32),
                    pltpu.VMEM((N_NODES_C, 256), jnp.float32)],
    compiler_params=_params(),
)

_tcm = pl.pallas_call(
    _tcmid_body,
    out_shape=[_f32(N_NODES_C, H_C), _f32(N_NODES_C, 2 * H_C)],
    compiler_params=_params(),
)

_tcx4 = pl.pallas_call(
    _tcx4_body,
    out_shape=[_f32(N_NODES_C, H_C)],
    compiler_params=_params(),
)

_tcatt = pl.pallas_call(
    _tcatt_body,
    out_shape=[_f32(N_NODES_C, 4 * H_C)],
    compiler_params=_params(),
)


from jax.experimental.pallas import tpu_sc as plsc

N_EDGES_C = 320000
NW = 32
NPW = 313
TRASH = NPW
EW_BIN = 8000
FLUSH = 2048
CAP = 324096
LW = 2048
GRP = LW // 16
NBUF = 4

_sc_mesh = plsc.VectorSubcoreMesh(core_axis_name="c", subcore_axis_name="s")


def _wid():
    return lax.axis_index("s") * 2 + lax.axis_index("c")


@functools.partial(
    pl.kernel,
    out_type=[jax.ShapeDtypeStruct((NW * CAP,), jnp.int32),
              jax.ShapeDtypeStruct((NW * 16,), jnp.int32)],
    mesh=_sc_mesh,
    scratch_types=[pltpu.VMEM((EW_BIN,), jnp.int32),
                   pltpu.VMEM((EW_BIN,), jnp.int32),
                   pltpu.VMEM((FLUSH + 32,), jnp.int32),
                   pltpu.VMEM((16,), jnp.int32)],
    compiler_params=pltpu.CompilerParams(needs_layout_passes=False),
)
def _sc_bin(src_hbm, dst_hbm, lists_hbm, counts_hbm, srcw, dstw, stage, cbuf):
    w = _wid()
    lo = w * NPW
    lbase = w * CAP
    lane = lax.broadcasted_iota(jnp.int32, (16,), 0)

    def window(t, carry):
        base = t * EW_BIN
        pltpu.sync_copy(src_hbm.at[pl.ds(base, EW_BIN)], srcw)
        pltpu.sync_copy(dst_hbm.at[pl.ds(base, EW_BIN)], dstw)

        def sort_group(k):
            s = srcw[pl.ds(k * 16, 16)]
            d = dstw[pl.ds(k * 16, 16)]
            ld = d - lo
            sel = ld.astype(jnp.uint32) < jnp.uint32(NPW)
            p = (ld << 16) | s
            key = jnp.where(sel, jnp.uint32(0), jnp.uint32(1))
            _, pv = plsc.sort_key_val(key, p)
            return pv, plsc.all_reduce_population_count(sel)[0]

        def group(k, car):
            off, flushes = car
            pv0, c0 = sort_group(2 * k)
            pv1, c1 = sort_group(2 * k + 1)
            stage[pl.ds(off, 16)] = pv0
            off1 = off + c0
            stage[pl.ds(off1, 16)] = pv1
            off = off1 + c1

            def do_flush(a):
                o, f = a
                pltpu.sync_copy(
                    stage.at[pl.ds(0, FLUSH)],
                    lists_hbm.at[pl.ds(lbase + f * FLUSH, FLUSH)])
                stage[pl.ds(0, 16)] = stage[pl.ds(FLUSH, 16)]
                stage[pl.ds(16, 16)] = stage[pl.ds(FLUSH + 16, 16)]
                return o - FLUSH, f + 1

            return lax.cond(off >= FLUSH, do_flush, lambda a: a,
                            (off, flushes))

        return lax.fori_loop(0, EW_BIN // 32, group, carry)

    off, flushes = lax.fori_loop(0, N_EDGES_C // EW_BIN, window,
                                 (jnp.int32(0), jnp.int32(0)))
    pltpu.sync_copy(stage.at[pl.ds(0, FLUSH)],
                    lists_hbm.at[pl.ds(lbase + flushes * FLUSH, FLUSH)])
    cbuf[...] = (flushes * FLUSH + off) + jnp.zeros((16,), jnp.int32)
    pltpu.sync_copy(cbuf, counts_hbm.at[pl.ds(w * 16, 16)])


@functools.partial(
    pl.kernel,
    out_type=jax.ShapeDtypeStruct((NW * NPW * H_C,), jnp.float32),
    mesh=_sc_mesh,
    scratch_types=[pltpu.VMEM(((GRP + NBUF) * 16,), jnp.int32)]
    + [pltpu.VMEM(((NPW + 1) * H_C,), jnp.float32)] * 4
    + [pltpu.VMEM((16,), jnp.int32)]
    + [pltpu.VMEM((16, H_C), jnp.float32)] * NBUF
    + [pltpu.VMEM((16,), jnp.int32)] * NBUF
    + [pltpu.SemaphoreType.DMA] * NBUF,
    compiler_params=pltpu.CompilerParams(needs_layout_passes=False,
                                         use_tc_tiling_on_sc=False),
)
def _sc_edge(B_hbm, lists_hbm, counts_hbm, out_hbm, listw, acc, acc2, acc3,
             acc4, cbuf, *scr):
    rows = list(scr[0:NBUF])
    ldbs = list(scr[NBUF:2 * NBUF])
    sems = list(scr[2 * NBUF:3 * NBUF])
    w = _wid()
    lbase = w * CAP
    iota = lax.broadcasted_iota(jnp.int32, (16,), 0)
    neg = jnp.full((16,), NEG_INF, jnp.float32)

    accs = [acc, acc2, acc3, acc4]

    def init(k, c):
        for a in accs:
            a[pl.ds(k * 16, 16)] = neg
        return c

    lax.fori_loop(0, (NPW + 1) * H_C // 16, init, 0)

    pltpu.sync_copy(counts_hbm.at[pl.ds(w * 16, 16)], cbuf)
    cnt = cbuf[pl.ds(0, 16)][0]

    def issue(b, g, t_base):
        gbase = g * 16
        p = listw[pl.ds(gbase, 16)]
        limit = jnp.where(gbase < LW, cnt, -1)
        valid = (t_base + gbase + iota) < limit
        s = jnp.where(valid, p & 0xFFFF, iota)
        ldbs[b][...] = jnp.where(valid, p >> 16, TRASH)
        pltpu.make_async_copy(B_hbm.at[s], rows[b], sems[b]).start()

    def process(b):
        pltpu.make_async_copy(B_hbm.at[iota], rows[b], sems[b]).wait()
        lv = ldbs[b][pl.ds(0, 16)] * H_C
        for e in range(16):
            base = lv[e]
            tgt = accs[e % 4]
            for cix in range(4):
                sl = pl.ds(base + cix * 16, 16)
                tgt[sl] = jnp.maximum(tgt[sl], rows[b][e, pl.ds(cix * 16, 16)])

    def window(t, c):
        t_base = t * LW
        pltpu.sync_copy(lists_hbm.at[pl.ds(lbase + t_base, LW)],
                        listw.at[pl.ds(0, LW)])
        for b in range(NBUF):
            issue(b, b, t_base)

        def chunk(cc, c2):
            for b in range(NBUF):
                process(b)
                issue(b, cc * NBUF + b + NBUF, t_base)
            return c2

        lax.fori_loop(0, GRP // NBUF, chunk, 0)
        for b in range(NBUF):
            process(b)
        return c

    lax.fori_loop(0, (cnt + LW - 1) // LW, window, 0)

    def merge(k, c):
        sl = pl.ds(k * 16, 16)
        acc[sl] = jnp.maximum(jnp.maximum(acc[sl], acc2[sl]),
                              jnp.maximum(acc3[sl], acc4[sl]))
        return c

    lax.fori_loop(0, NPW * H_C // 16, merge, 0)
    pltpu.sync_copy(acc.at[pl.ds(0, NPW * H_C)],
                    out_hbm.at[pl.ds(w * NPW * H_C, NPW * H_C)])


def kernel(x, edge_index, batch, W0, b0, Ws1, bs1, Ws2, bs2, Ws3, bs3,
           Wf1, bf1, Wf2, bf2, Wf3, bf3, We1, be1, We2, be2, We3, be3,
           We4, be4, Wg, bg):
    N = x.shape[0]
    batch2 = batch.reshape(N, 1).astype(jnp.int32)
    src = edge_index[0].astype(jnp.int32)
    dst = edge_index[1].astype(jnp.int32)

    def r2(b):
        return b.reshape(1, -1)

    lists, counts = _sc_bin(src, dst)

    def edge_max(AB):
        B = AB[:, H_C:]
        out = _sc_edge(B, lists, counts)
        return out.reshape(NW * NPW, H_C)[:N_NODES_C]

    (AB1,) = _tc1(x, batch2, W0, r2(b0), Ws1, r2(bs1), Ws2, r2(bs2),
                  Ws3, r2(bs3), Wf1, r2(bf1), Wf2, r2(bf2), Wf3, r2(bf3),
                  We1, r2(be1))

    m1 = edge_max(AB1)
    x1, AB2 = _tcm(m1, AB1, batch2, We2, r2(be2))
    m2 = edge_max(AB2)
    x2, AB3 = _tcm(m2, AB2, batch2, We3, r2(be3))
    m3 = edge_max(AB3)
    x3, AB4 = _tcm(m3, AB3, batch2, We4, r2(be4))
    m4 = edge_max(AB4)
    (x4,) = _tcx4(m4, AB4, batch2)

    xc = jnp.concatenate([x1, x2, x3, x4], axis=1)
    (tail,) = _tcatt(xc, batch2, Wg, r2(bg))
    return jnp.concatenate([xc, tail], axis=1)
